# baseline (speedup 1.0000x reference)
import math

import jax
import jax.numpy as jnp
from jax.experimental import pallas as pl
from jax.experimental.pallas import tpu as pltpu

N_HEADS = 16
N_KV_HEADS = 4
GROUP = N_HEADS // N_KV_HEADS
HEAD_DIM = 128
VSLOT = 2 * HEAD_DIM
ATTN_SCALE = 0.08838834764831845
QSCALE = ATTN_SCALE * math.log2(math.e)

BQ = 512

NEG_INF = float("-inf")


def _flash_body(q_ref, k_ref, v_ref, o_ref):
    i = pl.program_id(2)
    n_qblk = pl.num_programs(2)

    for c in range(n_qblk):
        @pl.when(i == c)
        def _(c=c):
            L = (c + 1) * BQ
            rows = jax.lax.broadcasted_iota(jnp.int32, (BQ, L), 0)
            cols = jax.lax.broadcasted_iota(jnp.int32, (BQ, L), 1)
            vis_mask = cols <= rows + (L - BQ)
            kb = k_ref[0, :L, :]
            vb = v_ref[0, :L, :]
            for hh in range(GROUP):
                qs = hh * HEAD_DIM
                q = (q_ref[0, :, qs:qs + HEAD_DIM] * QSCALE).astype(
                    jnp.bfloat16)
                s = jax.lax.dot_general(q, kb, (((1,), (1,)), ((), ())),
                                        preferred_element_type=jnp.float32)
                s16 = jnp.where(vis_mask, s.astype(jnp.bfloat16),
                                jnp.bfloat16(NEG_INF))
                p = jnp.exp2(s16)
                acc = jax.lax.dot_general(p, vb, (((1,), (0,)), ((), ())),
                                          preferred_element_type=jnp.float32)
                o_ref[0, :, qs:qs + HEAD_DIM] = (
                    acc[:, :HEAD_DIM] / acc[:, HEAD_DIM:HEAD_DIM + 1])


def kernel(query, key, value):
    B, S, QF = query.shape

    kb16 = key.astype(jnp.bfloat16)
    v4 = value.reshape(B, S, N_KV_HEADS, HEAD_DIM).astype(jnp.bfloat16)
    ones = jnp.ones((B, S, N_KV_HEADS, 1), jnp.bfloat16)
    zeros = jnp.zeros((B, S, N_KV_HEADS, HEAD_DIM - 1), jnp.bfloat16)
    vp = jnp.concatenate([v4, ones, zeros], axis=-1)
    vp = vp.reshape(B, S, N_KV_HEADS * VSLOT)

    return pl.pallas_call(
        _flash_body,
        grid=(B, N_KV_HEADS, S // BQ),
        in_specs=[
            pl.BlockSpec((1, BQ, GROUP * HEAD_DIM), lambda b, g, i: (b, i, g)),
            pl.BlockSpec((1, S, HEAD_DIM), lambda b, g, i: (b, 0, g)),
            pl.BlockSpec((1, S, VSLOT), lambda b, g, i: (b, 0, g)),
        ],
        out_specs=pl.BlockSpec((1, BQ, GROUP * HEAD_DIM),
                               lambda b, g, i: (b, i, g)),
        out_shape=jax.ShapeDtypeStruct((B, S, QF), jnp.float32),
        compiler_params=pltpu.CompilerParams(
            dimension_semantics=("parallel", "parallel", "arbitrary")),
    )(query, kb16, vp)


# speedup vs baseline: 1.3491x; 1.3491x over previous
import math

import jax
import jax.numpy as jnp
from jax.experimental import pallas as pl
from jax.experimental.pallas import tpu as pltpu

N_HEADS = 16
N_KV_HEADS = 4
GROUP = N_HEADS // N_KV_HEADS
HEAD_DIM = 128
VSLOT = 2 * HEAD_DIM
ATTN_SCALE = 0.08838834764831845
QSCALE = ATTN_SCALE * math.log2(math.e)

BQ = 512

NEG_INF = float("-inf")


def _flash_body(q_ref, k_ref, v_ref, o_ref, v_scr):
    i = pl.program_id(2)
    n_qblk = pl.num_programs(2)
    S = k_ref.shape[1]

    @pl.when(i == 0)
    def _():
        v_scr[:, :HEAD_DIM] = v_ref[0, :, :].astype(jnp.bfloat16)
        lane = jax.lax.broadcasted_iota(jnp.int32, (S, HEAD_DIM), 1)
        v_scr[:, HEAD_DIM:] = jnp.where(lane == 0, jnp.float32(1),
                                        jnp.float32(0)).astype(jnp.bfloat16)

    for c in range(n_qblk):
        @pl.when(i == c)
        def _(c=c):
            L = (c + 1) * BQ
            rows = jax.lax.broadcasted_iota(jnp.int32, (BQ, L), 0)
            cols = jax.lax.broadcasted_iota(jnp.int32, (BQ, L), 1)
            vis_mask = cols <= rows + (L - BQ)
            kb = k_ref[0, :L, :].astype(jnp.bfloat16)
            vb = v_scr[:L, :]
            for hh in range(GROUP):
                qs = hh * HEAD_DIM
                q = (q_ref[0, :, qs:qs + HEAD_DIM] * QSCALE).astype(
                    jnp.bfloat16)
                s = jax.lax.dot_general(q, kb, (((1,), (1,)), ((), ())),
                                        preferred_element_type=jnp.float32)
                s = jnp.where(vis_mask, s, NEG_INF)
                p = jnp.exp2(s).astype(jnp.bfloat16)
                acc = jax.lax.dot_general(p, vb, (((1,), (0,)), ((), ())),
                                          preferred_element_type=jnp.float32)
                o_ref[0, :, qs:qs + HEAD_DIM] = (
                    acc[:, :HEAD_DIM] / acc[:, HEAD_DIM:HEAD_DIM + 1])


def kernel(query, key, value):
    B, S, QF = query.shape

    return pl.pallas_call(
        _flash_body,
        grid=(B, N_KV_HEADS, S // BQ),
        in_specs=[
            pl.BlockSpec((1, BQ, GROUP * HEAD_DIM), lambda b, g, i: (b, i, g)),
            pl.BlockSpec((1, S, HEAD_DIM), lambda b, g, i: (b, 0, g)),
            pl.BlockSpec((1, S, HEAD_DIM), lambda b, g, i: (b, 0, g)),
        ],
        out_specs=pl.BlockSpec((1, BQ, GROUP * HEAD_DIM),
                               lambda b, g, i: (b, i, g)),
        out_shape=jax.ShapeDtypeStruct((B, S, QF), jnp.float32),
        scratch_shapes=[pltpu.VMEM((S, VSLOT), jnp.bfloat16)],
        compiler_params=pltpu.CompilerParams(
            dimension_semantics=("parallel", "parallel", "arbitrary")),
    )(query, key, value)


# speedup vs baseline: 1.4323x; 1.0617x over previous
import math

import jax
import jax.numpy as jnp
from jax.experimental import pallas as pl
from jax.experimental.pallas import tpu as pltpu

N_HEADS = 16
N_KV_HEADS = 4
GROUP = N_HEADS // N_KV_HEADS
KV_PER_PROG = 2
HEADS_PER_PROG = KV_PER_PROG * GROUP
HEAD_DIM = 128
VSLOT = 2 * HEAD_DIM
ATTN_SCALE = 0.08838834764831845
QSCALE = ATTN_SCALE * math.log2(math.e)

BQ = 512

NEG_INF = float("-inf")


def _flash_body(q_ref, k_ref, v_ref, o_ref, v_scr):
    i = pl.program_id(2)
    n_qblk = pl.num_programs(2)
    S = k_ref.shape[1]

    @pl.when(i == 0)
    def _():
        lane = jax.lax.broadcasted_iota(jnp.int32, (S, HEAD_DIM), 1)
        ones_col = jnp.where(lane == 0, jnp.float32(1),
                             jnp.float32(0)).astype(jnp.bfloat16)
        for kv in range(KV_PER_PROG):
            v_scr[:, kv * VSLOT:kv * VSLOT + HEAD_DIM] = (
                v_ref[0, :, kv * HEAD_DIM:(kv + 1) * HEAD_DIM].astype(
                    jnp.bfloat16))
            v_scr[:, kv * VSLOT + HEAD_DIM:(kv + 1) * VSLOT] = ones_col

    for c in range(n_qblk):
        @pl.when(i == c)
        def _(c=c):
            L = (c + 1) * BQ
            rows = jax.lax.broadcasted_iota(jnp.int32, (BQ, L), 0)
            cols = jax.lax.broadcasted_iota(jnp.int32, (BQ, L), 1)
            vis_mask = cols <= rows + (L - BQ)
            kb16 = k_ref[0, :L, :].astype(jnp.bfloat16)
            for hh in range(HEADS_PER_PROG):
                kv = hh // GROUP
                qs = hh * HEAD_DIM
                kb = kb16[:, kv * HEAD_DIM:(kv + 1) * HEAD_DIM]
                vb = v_scr[:L, kv * VSLOT:(kv + 1) * VSLOT]
                q = (q_ref[0, :, qs:qs + HEAD_DIM] * QSCALE).astype(
                    jnp.bfloat16)
                s = jax.lax.dot_general(q, kb, (((1,), (1,)), ((), ())),
                                        preferred_element_type=jnp.float32)
                s = jnp.where(vis_mask, s, NEG_INF)
                p = jnp.exp2(s).astype(jnp.bfloat16)
                acc = jax.lax.dot_general(p, vb, (((1,), (0,)), ((), ())),
                                          preferred_element_type=jnp.float32)
                o_ref[0, :, qs:qs + HEAD_DIM] = (
                    acc[:, :HEAD_DIM] / acc[:, HEAD_DIM:HEAD_DIM + 1])


def kernel(query, key, value):
    B, S, QF = query.shape

    return pl.pallas_call(
        _flash_body,
        grid=(B, N_KV_HEADS // KV_PER_PROG, S // BQ),
        in_specs=[
            pl.BlockSpec((1, BQ, HEADS_PER_PROG * HEAD_DIM),
                         lambda b, g, i: (b, i, g)),
            pl.BlockSpec((1, S, KV_PER_PROG * HEAD_DIM),
                         lambda b, g, i: (b, 0, g)),
            pl.BlockSpec((1, S, KV_PER_PROG * HEAD_DIM),
                         lambda b, g, i: (b, 0, g)),
        ],
        out_specs=pl.BlockSpec((1, BQ, HEADS_PER_PROG * HEAD_DIM),
                               lambda b, g, i: (b, i, g)),
        out_shape=jax.ShapeDtypeStruct((B, S, QF), jnp.float32),
        scratch_shapes=[pltpu.VMEM((S, KV_PER_PROG * VSLOT), jnp.bfloat16)],
        compiler_params=pltpu.CompilerParams(
            dimension_semantics=("parallel", "parallel", "arbitrary")),
    )(query, key, value)
